# baseline (speedup 1.0000x reference)
import functools

import jax
import jax.numpy as jnp
from jax import lax
from jax.experimental import pallas as pl
from jax.experimental.pallas import tpu as pltpu
from jax.experimental.pallas import tpu_sc as plsc

N_NODES = 10000
LATENT = 32
N_EDGES = 160000
NW = 32
GW = 128
GROUPS = 40
EP = NW * GROUPS * GW
BROWS = 10016
SPAD = 4 * BROWS
ZROWS = 256


def _sc_gather_scatter():
    mesh = plsc.VectorSubcoreMesh(core_axis_name="c", subcore_axis_name="s")

    @functools.partial(
        pl.kernel,
        mesh=mesh,
        compiler_params=pltpu.CompilerParams(use_tc_tiling_on_sc=False),
        out_type=jax.ShapeDtypeStruct((2, SPAD, LATENT), jnp.float32),
        scratch_types=[
            pltpu.VMEM((GROUPS, GW), jnp.int32),
            pltpu.VMEM((GROUPS, GW), jnp.int32),
            pltpu.VMEM((4, GW, LATENT), jnp.float32),
            pltpu.VMEM((ZROWS, LATENT), jnp.float32),
            pltpu.VMEM_SHARED((SPAD, LATENT), jnp.float32),
            [pltpu.SemaphoreType.DMA] * 4,
            [pltpu.SemaphoreType.DMA] * 4,
        ],
    )
    def sc_fn(table, src, comb, out, src_v, comb_v,
              rows, zbuf, acc, gsem, ssem):
        cid = lax.axis_index("c")
        sid = lax.axis_index("s")
        wid = sid * 2 + cid

        pltpu.sync_copy(src.at[pl.ds(wid * GROUPS, GROUPS)], src_v)
        pltpu.sync_copy(comb.at[pl.ds(wid * GROUPS, GROUPS)], comb_v)

        z16 = jnp.zeros((16,), jnp.float32)

        def zrow(r, carry):
            zbuf[r, pl.ds(0, 16)] = z16
            zbuf[r, pl.ds(16, 16)] = z16
            return carry

        lax.fori_loop(0, ZROWS, zrow, 0, unroll=8)
        for k in range(9):
            pltpu.sync_copy(zbuf, acc.at[pl.ds(sid * 2504 + k * ZROWS, ZROWS)])
        pltpu.sync_copy(zbuf.at[pl.ds(0, 200)],
                        acc.at[pl.ds(sid * 2504 + 2304, 200)])

        plsc.subcore_barrier()

        def body(g, carry):
            for b in range(4):
                @pl.when(g + b >= 4)
                def _():
                    pltpu.make_async_copy(
                        rows.at[b], acc.at[comb_v.at[g + b - 4]],
                        ssem[b]).wait()
                pltpu.async_copy(table.at[src_v.at[g + b]], rows.at[b], gsem[b])
            for b in range(4):
                pltpu.make_async_copy(table.at[src_v.at[g + b]], rows.at[b],
                                      gsem[b]).wait()
                pltpu.async_copy(rows.at[b], acc.at[comb_v.at[g + b]], ssem[b],
                                 add=True)
            return carry

        lax.fori_loop(0, GROUPS // 4, lambda i, c: body(i * 4, c), 0)
        for b in range(4):
            pltpu.make_async_copy(rows.at[b], acc.at[comb_v.at[GROUPS - 4 + b]],
                                  ssem[b]).wait()

        plsc.subcore_barrier()
        pltpu.sync_copy(acc.at[pl.ds(sid * 2504, 2504)],
                        out.at[cid, pl.ds(sid * 2504, 2504)])

    return sc_fn


def _tc_input_proj(nb):
    B = N_NODES // nb
    GB = (EP // GW) // nb

    def body(x_r, wl_r, bl_r, wr_r, wz_r, wn_r, br_r, bz_r, bn_r,
             a0_r, a1_r, a2_r, a3_r, dst_r,
             h0_r, h0b_r, ghr_r, ghz_r, ghn_r, comb_r):
        h0 = jnp.maximum(
            jnp.dot(x_r[...], wl_r[...], preferred_element_type=jnp.float32)
            + bl_r[...], 0.0)
        h0_r[...] = h0
        h0b_r[...] = h0.astype(jnp.bfloat16).astype(jnp.float32)
        ghr_r[...] = jnp.dot(h0, wr_r[...], preferred_element_type=jnp.float32) + br_r[...]
        ghz_r[...] = jnp.dot(h0, wz_r[...], preferred_element_type=jnp.float32) + bz_r[...]
        ghn_r[...] = jnp.dot(h0, wn_r[...], preferred_element_type=jnp.float32) + bn_r[...]
        best = a0_r[...]
        b = jnp.zeros_like(dst_r[...])
        for k, ak_r in ((1, a1_r), (2, a2_r), (3, a3_r)):
            ak = ak_r[...]
            gt = ak > best
            best = jnp.where(gt, ak, best)
            b = jnp.where(gt, k, b)
        comb_r[...] = b * BROWS + dst_r[...]

    full = lambda s: pl.BlockSpec(s, lambda i: tuple(0 for _ in s))
    row = lambda w: pl.BlockSpec((B, w), lambda i: (i, 0))
    erow = pl.BlockSpec((GB, GW), lambda i: (i, 0))
    return pl.pallas_call(
        body,
        grid=(nb,),
        in_specs=[row(128), full((128, LATENT)), full((1, LATENT)),
                  full((LATENT, LATENT)), full((LATENT, LATENT)), full((LATENT, LATENT)),
                  full((1, LATENT)), full((1, LATENT)), full((1, LATENT)),
                  erow, erow, erow, erow, erow],
        out_specs=[row(LATENT)] * 5 + [erow],
        out_shape=[jax.ShapeDtypeStruct((N_NODES, LATENT), jnp.float32)] * 5
        + [jax.ShapeDtypeStruct((EP // GW, GW), jnp.int32)],
    )


def _tc_gru(nb, final):
    B = N_NODES // nb

    def body(*refs):
        (p_r, h0_r, ghr_r, ghz_r, ghn_r, wcat_r,
         wir_r, wiz_r, win_r, bir_r, biz_r, bin_r) = refs[:12]
        if final:
            wmu_r, bmu_r, wlv_r, blv_r = refs[12:16]
            outs = refs[16:]
        else:
            outs = refs[12:]
        agg = jnp.zeros((B, LATENT), jnp.float32)
        for b in range(4):
            agg = agg + jnp.dot(p_r[0, b] + p_r[1, b], wcat_r[b],
                                preferred_element_type=jnp.float32,
                                precision=jax.lax.Precision.HIGHEST)
        gir = jnp.dot(agg, wir_r[...], preferred_element_type=jnp.float32) + bir_r[...]
        giz = jnp.dot(agg, wiz_r[...], preferred_element_type=jnp.float32) + biz_r[...]
        gin = jnp.dot(agg, win_r[...], preferred_element_type=jnp.float32) + bin_r[...]
        r = jax.nn.sigmoid(gir + ghr_r[...])
        z = jax.nn.sigmoid(giz + ghz_r[...])
        n = jnp.tanh(gin + r * ghn_r[...])
        h0 = h0_r[...]
        m = jnp.maximum((1.0 - z) * n + z * h0, 0.0)
        if final:
            outs[0][...] = jnp.dot(m, wmu_r[...], preferred_element_type=jnp.float32) + bmu_r[...]
            outs[1][...] = jnp.dot(m, wlv_r[...], preferred_element_type=jnp.float32) + blv_r[...]
        else:
            outs[0][...] = m.astype(jnp.bfloat16).astype(jnp.float32)

    full = lambda s: pl.BlockSpec(s, lambda i: tuple(0 for _ in s))
    row = lambda w: pl.BlockSpec((B, w), lambda i: (i, 0))
    pspec = pl.BlockSpec((2, 4, B, LATENT), lambda i: (0, 0, i, 0))
    in_specs = [pspec, row(LATENT), row(LATENT), row(LATENT), row(LATENT),
                full((4, LATENT, LATENT)),
                full((LATENT, LATENT)), full((LATENT, LATENT)), full((LATENT, LATENT)),
                full((1, LATENT)), full((1, LATENT)), full((1, LATENT))]
    if final:
        in_specs += [full((LATENT, LATENT)), full((1, LATENT)),
                     full((LATENT, LATENT)), full((1, LATENT))]
        out_specs = [row(LATENT), row(LATENT)]
        out_shape = [jax.ShapeDtypeStruct((N_NODES, LATENT), jnp.float32)] * 2
    else:
        out_specs = [row(LATENT)]
        out_shape = [jax.ShapeDtypeStruct((N_NODES, LATENT), jnp.float32)]
    return pl.pallas_call(
        body, grid=(nb,), in_specs=in_specs, out_specs=out_specs,
        out_shape=out_shape)


def kernel(x, edge_index, edge_attr, W_lin, b_lin, gnn_weight,
           gru_w_ih, gru_w_hh, gru_b_ih, gru_b_hh,
           W_mu, b_mu, W_lv, b_lv):
    ei = edge_index.astype(jnp.int32)
    src = jnp.pad(ei[0], (0, EP - N_EDGES)).reshape(EP // GW, GW)
    dst = jnp.pad(ei[1], (0, EP - N_EDGES),
                  constant_values=N_NODES).reshape(EP // GW, GW)
    acol = [jnp.pad(edge_attr[:, k], (0, EP - N_EDGES)).reshape(EP // GW, GW)
            for k in range(4)]

    row1 = lambda v: v.reshape(1, LATENT)
    wl = W_lin.T
    whh = [gru_w_hh[k * LATENT:(k + 1) * LATENT].T for k in range(3)]
    bhh = [row1(gru_b_hh[k * LATENT:(k + 1) * LATENT]) for k in range(3)]
    wih = [gru_w_ih[k * LATENT:(k + 1) * LATENT].T for k in range(3)]
    bih = [row1(gru_b_ih[k * LATENT:(k + 1) * LATENT]) for k in range(3)]
    wcat = [2.0 * gnn_weight[i].astype(jnp.bfloat16).astype(jnp.float32)
            .transpose(0, 2, 1)
            for i in range(gnn_weight.shape[0])]

    nb = 10
    h0, h0b, ghr, ghz, ghn, comb = _tc_input_proj(nb)(
        x, wl, row1(b_lin), whh[0], whh[1], whh[2], bhh[0], bhh[1], bhh[2],
        acol[0], acol[1], acol[2], acol[3], dst)

    sc = _sc_gather_scatter()
    gru = _tc_gru(nb, final=False)
    gru_final = _tc_gru(nb, final=True)

    m = h0b
    num_layers = gnn_weight.shape[0]
    for i in range(num_layers):
        P = sc(m, src, comb)
        pm = P.reshape(2, 4, BROWS, LATENT)
        args = (pm, h0, ghr, ghz, ghn, wcat[i],
                wih[0], wih[1], wih[2], bih[0], bih[1], bih[2])
        if i == num_layers - 1:
            mu, lv = gru_final(*args, W_mu.T, row1(b_mu), W_lv.T, row1(b_lv))
        else:
            (m,) = gru(*args)
    return (mu, lv)


# speedup vs baseline: 1.3216x; 1.3216x over previous
import functools

import jax
import jax.numpy as jnp
from jax import lax
from jax.experimental import pallas as pl
from jax.experimental.pallas import tpu as pltpu
from jax.experimental.pallas import tpu_sc as plsc

N_NODES = 10000
LATENT = 32
N_EDGES = 160000
NW = 32
GW = 128
GROUPS = 40
EP = NW * GROUPS * GW
BROWS = 10016
SPAD = 4 * BROWS
ZROWS = 256


def _sc_gather_scatter():
    mesh = plsc.VectorSubcoreMesh(core_axis_name="c", subcore_axis_name="s")

    @functools.partial(
        pl.kernel,
        mesh=mesh,
        compiler_params=pltpu.CompilerParams(use_tc_tiling_on_sc=False),
        out_type=jax.ShapeDtypeStruct((2, SPAD, LATENT), jnp.float32),
        scratch_types=[
            pltpu.VMEM((GROUPS, GW), jnp.int32),
            pltpu.VMEM((GROUPS, GW), jnp.int32),
            pltpu.VMEM((4, GW, LATENT), jnp.float32),
            pltpu.VMEM((ZROWS, LATENT), jnp.float32),
            pltpu.VMEM_SHARED((SPAD, LATENT), jnp.float32),
            [pltpu.SemaphoreType.DMA] * 4,
            [pltpu.SemaphoreType.DMA] * 4,
        ],
    )
    def sc_fn(table, src, comb, out, src_v, comb_v,
              rows, zbuf, acc, gsem, ssem):
        cid = lax.axis_index("c")
        sid = lax.axis_index("s")
        wid = sid * 2 + cid

        pltpu.sync_copy(src.at[pl.ds(wid * GROUPS, GROUPS)], src_v)
        pltpu.sync_copy(comb.at[pl.ds(wid * GROUPS, GROUPS)], comb_v)

        z16 = jnp.zeros((16,), jnp.float32)

        def zrow(r, carry):
            zbuf[r, pl.ds(0, 16)] = z16
            zbuf[r, pl.ds(16, 16)] = z16
            return carry

        lax.fori_loop(0, ZROWS, zrow, 0, unroll=8)
        for k in range(9):
            pltpu.sync_copy(zbuf, acc.at[pl.ds(sid * 2504 + k * ZROWS, ZROWS)])
        pltpu.sync_copy(zbuf.at[pl.ds(0, 200)],
                        acc.at[pl.ds(sid * 2504 + 2304, 200)])

        plsc.subcore_barrier()

        def body(g, carry):
            for b in range(4):
                @pl.when(g + b >= 4)
                def _():
                    pltpu.make_async_copy(
                        rows.at[b], acc.at[comb_v.at[g + b - 4]],
                        ssem[b]).wait()
                pltpu.async_copy(table.at[src_v.at[g + b]], rows.at[b], gsem[b])
            for b in range(4):
                pltpu.make_async_copy(table.at[src_v.at[g + b]], rows.at[b],
                                      gsem[b]).wait()
                pltpu.async_copy(rows.at[b], acc.at[comb_v.at[g + b]], ssem[b],
                                 add=True)
            return carry

        lax.fori_loop(0, GROUPS // 4, lambda i, c: body(i * 4, c), 0)
        for b in range(4):
            pltpu.make_async_copy(rows.at[b], acc.at[comb_v.at[GROUPS - 4 + b]],
                                  ssem[b]).wait()

        plsc.subcore_barrier()
        pltpu.sync_copy(acc.at[pl.ds(sid * 2504, 2504)],
                        out.at[cid, pl.ds(sid * 2504, 2504)])

    return sc_fn


def _tc_input_proj(nb):
    B = N_NODES // nb
    GB = (EP // GW) // nb

    def body(x_r, wl_r, bl_r, wr_r, wz_r, wn_r, br_r, bz_r, bn_r,
             a0_r, a1_r, a2_r, a3_r, dst_r,
             h0_r, h0b_r, ghr_r, ghz_r, ghn_r, comb_r):
        h0 = jnp.maximum(
            jnp.dot(x_r[...], wl_r[...], preferred_element_type=jnp.float32)
            + bl_r[...], 0.0)
        h0_r[...] = h0
        h0b_r[...] = h0.astype(jnp.bfloat16).astype(jnp.float32)
        ghr_r[...] = jnp.dot(h0, wr_r[...], preferred_element_type=jnp.float32) + br_r[...]
        ghz_r[...] = jnp.dot(h0, wz_r[...], preferred_element_type=jnp.float32) + bz_r[...]
        ghn_r[...] = jnp.dot(h0, wn_r[...], preferred_element_type=jnp.float32) + bn_r[...]
        best = a0_r[...]
        b = jnp.zeros_like(dst_r[...])
        for k, ak_r in ((1, a1_r), (2, a2_r), (3, a3_r)):
            ak = ak_r[...]
            gt = ak > best
            best = jnp.where(gt, ak, best)
            b = jnp.where(gt, k, b)
        comb_r[...] = dst_r[...] * 4 + b

    full = lambda s: pl.BlockSpec(s, lambda i: tuple(0 for _ in s))
    row = lambda w: pl.BlockSpec((B, w), lambda i: (i, 0))
    erow = pl.BlockSpec((GB, GW), lambda i: (i, 0))
    return pl.pallas_call(
        body,
        grid=(nb,),
        in_specs=[row(128), full((128, LATENT)), full((1, LATENT)),
                  full((LATENT, LATENT)), full((LATENT, LATENT)), full((LATENT, LATENT)),
                  full((1, LATENT)), full((1, LATENT)), full((1, LATENT)),
                  erow, erow, erow, erow, erow],
        out_specs=[row(LATENT)] * 5 + [erow],
        out_shape=[jax.ShapeDtypeStruct((N_NODES, LATENT), jnp.float32)] * 5
        + [jax.ShapeDtypeStruct((EP // GW, GW), jnp.int32)],
    )


def _tc_gru(nb, final):
    B = N_NODES // nb

    def body(*refs):
        (p_r, h0_r, ghr_r, ghz_r, ghn_r, wcat_r,
         wir_r, wiz_r, win_r, bir_r, biz_r, bin_r) = refs[:12]
        if final:
            wmu_r, bmu_r, wlv_r, blv_r = refs[12:16]
            outs = refs[16:]
        else:
            outs = refs[12:]
        agg = jnp.dot(p_r[0] + p_r[1], wcat_r[...],
                      preferred_element_type=jnp.float32,
                      precision=jax.lax.Precision.HIGHEST)
        gir = jnp.dot(agg, wir_r[...], preferred_element_type=jnp.float32) + bir_r[...]
        giz = jnp.dot(agg, wiz_r[...], preferred_element_type=jnp.float32) + biz_r[...]
        gin = jnp.dot(agg, win_r[...], preferred_element_type=jnp.float32) + bin_r[...]
        r = jax.nn.sigmoid(gir + ghr_r[...])
        z = jax.nn.sigmoid(giz + ghz_r[...])
        n = jnp.tanh(gin + r * ghn_r[...])
        h0 = h0_r[...]
        m = jnp.maximum((1.0 - z) * n + z * h0, 0.0)
        if final:
            outs[0][...] = jnp.dot(m, wmu_r[...], preferred_element_type=jnp.float32) + bmu_r[...]
            outs[1][...] = jnp.dot(m, wlv_r[...], preferred_element_type=jnp.float32) + blv_r[...]
        else:
            outs[0][...] = m.astype(jnp.bfloat16).astype(jnp.float32)

    full = lambda s: pl.BlockSpec(s, lambda i: tuple(0 for _ in s))
    row = lambda w: pl.BlockSpec((B, w), lambda i: (i, 0))
    pspec = pl.BlockSpec((2, B, 4 * LATENT), lambda i: (0, i, 0))
    in_specs = [pspec, row(LATENT), row(LATENT), row(LATENT), row(LATENT),
                full((4 * LATENT, LATENT)),
                full((LATENT, LATENT)), full((LATENT, LATENT)), full((LATENT, LATENT)),
                full((1, LATENT)), full((1, LATENT)), full((1, LATENT))]
    if final:
        in_specs += [full((LATENT, LATENT)), full((1, LATENT)),
                     full((LATENT, LATENT)), full((1, LATENT))]
        out_specs = [row(LATENT), row(LATENT)]
        out_shape = [jax.ShapeDtypeStruct((N_NODES, LATENT), jnp.float32)] * 2
    else:
        out_specs = [row(LATENT)]
        out_shape = [jax.ShapeDtypeStruct((N_NODES, LATENT), jnp.float32)]
    return pl.pallas_call(
        body, grid=(nb,), in_specs=in_specs, out_specs=out_specs,
        out_shape=out_shape)


def kernel(x, edge_index, edge_attr, W_lin, b_lin, gnn_weight,
           gru_w_ih, gru_w_hh, gru_b_ih, gru_b_hh,
           W_mu, b_mu, W_lv, b_lv):
    ei = edge_index.astype(jnp.int32)
    src = jnp.pad(ei[0], (0, EP - N_EDGES)).reshape(EP // GW, GW)
    dst = jnp.pad(ei[1], (0, EP - N_EDGES),
                  constant_values=N_NODES).reshape(EP // GW, GW)
    acol = [jnp.pad(edge_attr[:, k], (0, EP - N_EDGES)).reshape(EP // GW, GW)
            for k in range(4)]

    row1 = lambda v: v.reshape(1, LATENT)
    wl = W_lin.T
    whh = [gru_w_hh[k * LATENT:(k + 1) * LATENT].T for k in range(3)]
    bhh = [row1(gru_b_hh[k * LATENT:(k + 1) * LATENT]) for k in range(3)]
    wih = [gru_w_ih[k * LATENT:(k + 1) * LATENT].T for k in range(3)]
    bih = [row1(gru_b_ih[k * LATENT:(k + 1) * LATENT]) for k in range(3)]
    wcat = [2.0 * gnn_weight[i].astype(jnp.bfloat16).astype(jnp.float32)
            .transpose(0, 2, 1).reshape(4 * LATENT, LATENT)
            for i in range(gnn_weight.shape[0])]

    nb = 10
    h0, h0b, ghr, ghz, ghn, comb = _tc_input_proj(nb)(
        x, wl, row1(b_lin), whh[0], whh[1], whh[2], bhh[0], bhh[1], bhh[2],
        acol[0], acol[1], acol[2], acol[3], dst)

    sc = _sc_gather_scatter()
    gru = _tc_gru(nb, final=False)
    gru_final = _tc_gru(nb, final=True)

    m = h0b
    num_layers = gnn_weight.shape[0]
    for i in range(num_layers):
        P = sc(m, src, comb)
        pm = P.reshape(2, BROWS, 4 * LATENT)
        args = (pm, h0, ghr, ghz, ghn, wcat[i],
                wih[0], wih[1], wih[2], bih[0], bih[1], bih[2])
        if i == num_layers - 1:
            mu, lv = gru_final(*args, W_mu.T, row1(b_mu), W_lv.T, row1(b_lv))
        else:
            (m,) = gru(*args)
    return (mu, lv)


# speedup vs baseline: 1.3611x; 1.0298x over previous
import functools

import jax
import jax.numpy as jnp
from jax import lax
from jax.experimental import pallas as pl
from jax.experimental.pallas import tpu as pltpu
from jax.experimental.pallas import tpu_sc as plsc

N_NODES = 10000
LATENT = 32
N_EDGES = 160000
NW = 32
GW = 128
GROUPS = 40
EP = NW * GROUPS * GW
BROWS = 10016
SPAD = 4 * BROWS
ZROWS = 128


def _sc_gather_scatter():
    mesh = plsc.VectorSubcoreMesh(core_axis_name="c", subcore_axis_name="s")

    @functools.partial(
        pl.kernel,
        mesh=mesh,
        compiler_params=pltpu.CompilerParams(use_tc_tiling_on_sc=False),
        out_type=jax.ShapeDtypeStruct((2, SPAD, LATENT), jnp.float32),
        scratch_types=[
            pltpu.VMEM((GROUPS, GW), jnp.int32),
            pltpu.VMEM((GROUPS, GW), jnp.int32),
            pltpu.VMEM((8, GW, LATENT), jnp.float32),
            pltpu.VMEM((ZROWS, LATENT), jnp.float32),
            pltpu.VMEM_SHARED((SPAD, LATENT), jnp.float32),
            [pltpu.SemaphoreType.DMA] * 8,
            [pltpu.SemaphoreType.DMA] * 8,
        ],
    )
    def sc_fn(table, src, comb, out, src_v, comb_v,
              rows, zbuf, acc, gsem, ssem):
        cid = lax.axis_index("c")
        sid = lax.axis_index("s")
        wid = sid * 2 + cid

        pltpu.sync_copy(src.at[pl.ds(wid * GROUPS, GROUPS)], src_v)
        pltpu.sync_copy(comb.at[pl.ds(wid * GROUPS, GROUPS)], comb_v)

        z16 = jnp.zeros((16,), jnp.float32)

        def zrow(r, carry):
            zbuf[r, pl.ds(0, 16)] = z16
            zbuf[r, pl.ds(16, 16)] = z16
            return carry

        lax.fori_loop(0, ZROWS, zrow, 0, unroll=8)
        for k in range(19):
            pltpu.sync_copy(zbuf, acc.at[pl.ds(sid * 2504 + k * ZROWS, ZROWS)])
        pltpu.sync_copy(zbuf.at[pl.ds(0, 72)],
                        acc.at[pl.ds(sid * 2504 + 2432, 72)])

        plsc.subcore_barrier()

        ND = 8

        def body(g, carry):
            for b in range(ND):
                @pl.when(g + b >= ND)
                def _():
                    pltpu.make_async_copy(
                        rows.at[b], acc.at[comb_v.at[g + b - ND]],
                        ssem[b]).wait()
                pltpu.async_copy(table.at[src_v.at[g + b]], rows.at[b], gsem[b])
            for b in range(ND):
                pltpu.make_async_copy(table.at[src_v.at[g + b]], rows.at[b],
                                      gsem[b]).wait()
                pltpu.async_copy(rows.at[b], acc.at[comb_v.at[g + b]], ssem[b],
                                 add=True)
            return carry

        lax.fori_loop(0, GROUPS // ND, lambda i, c: body(i * ND, c), 0)
        for b in range(ND):
            pltpu.make_async_copy(rows.at[b], acc.at[comb_v.at[GROUPS - ND + b]],
                                  ssem[b]).wait()

        plsc.subcore_barrier()
        pltpu.sync_copy(acc.at[pl.ds(sid * 2504, 2504)],
                        out.at[cid, pl.ds(sid * 2504, 2504)])

    return sc_fn


def _tc_input_proj(nb):
    B = N_NODES // nb
    GB = (EP // GW) // nb

    def body(x_r, wl_r, bl_r, wr_r, wz_r, wn_r, br_r, bz_r, bn_r,
             a0_r, a1_r, a2_r, a3_r, dst_r,
             h0_r, h0b_r, ghr_r, ghz_r, ghn_r, comb_r):
        h0 = jnp.maximum(
            jnp.dot(x_r[...], wl_r[...], preferred_element_type=jnp.float32)
            + bl_r[...], 0.0)
        h0_r[...] = h0
        h0b_r[...] = h0.astype(jnp.bfloat16).astype(jnp.float32)
        ghr_r[...] = jnp.dot(h0, wr_r[...], preferred_element_type=jnp.float32) + br_r[...]
        ghz_r[...] = jnp.dot(h0, wz_r[...], preferred_element_type=jnp.float32) + bz_r[...]
        ghn_r[...] = jnp.dot(h0, wn_r[...], preferred_element_type=jnp.float32) + bn_r[...]
        best = a0_r[...]
        b = jnp.zeros_like(dst_r[...])
        for k, ak_r in ((1, a1_r), (2, a2_r), (3, a3_r)):
            ak = ak_r[...]
            gt = ak > best
            best = jnp.where(gt, ak, best)
            b = jnp.where(gt, k, b)
        comb_r[...] = dst_r[...] * 4 + b

    full = lambda s: pl.BlockSpec(s, lambda i: tuple(0 for _ in s))
    row = lambda w: pl.BlockSpec((B, w), lambda i: (i, 0))
    erow = pl.BlockSpec((GB, GW), lambda i: (i, 0))
    return pl.pallas_call(
        body,
        grid=(nb,),
        in_specs=[row(128), full((128, LATENT)), full((1, LATENT)),
                  full((LATENT, LATENT)), full((LATENT, LATENT)), full((LATENT, LATENT)),
                  full((1, LATENT)), full((1, LATENT)), full((1, LATENT)),
                  erow, erow, erow, erow, erow],
        out_specs=[row(LATENT)] * 5 + [erow],
        out_shape=[jax.ShapeDtypeStruct((N_NODES, LATENT), jnp.float32)] * 5
        + [jax.ShapeDtypeStruct((EP // GW, GW), jnp.int32)],
    )


def _tc_gru(nb, final):
    B = N_NODES // nb

    def body(*refs):
        (p_r, h0_r, ghr_r, ghz_r, ghn_r, wcat_r,
         wir_r, wiz_r, win_r, bir_r, biz_r, bin_r) = refs[:12]
        if final:
            wmu_r, bmu_r, wlv_r, blv_r = refs[12:16]
            outs = refs[16:]
        else:
            outs = refs[12:]
        agg = jnp.dot(p_r[0] + p_r[1], wcat_r[...],
                      preferred_element_type=jnp.float32,
                      precision=jax.lax.Precision.HIGHEST)
        gir = jnp.dot(agg, wir_r[...], preferred_element_type=jnp.float32) + bir_r[...]
        giz = jnp.dot(agg, wiz_r[...], preferred_element_type=jnp.float32) + biz_r[...]
        gin = jnp.dot(agg, win_r[...], preferred_element_type=jnp.float32) + bin_r[...]
        r = jax.nn.sigmoid(gir + ghr_r[...])
        z = jax.nn.sigmoid(giz + ghz_r[...])
        n = jnp.tanh(gin + r * ghn_r[...])
        h0 = h0_r[...]
        m = jnp.maximum((1.0 - z) * n + z * h0, 0.0)
        if final:
            outs[0][...] = jnp.dot(m, wmu_r[...], preferred_element_type=jnp.float32) + bmu_r[...]
            outs[1][...] = jnp.dot(m, wlv_r[...], preferred_element_type=jnp.float32) + blv_r[...]
        else:
            outs[0][...] = m.astype(jnp.bfloat16).astype(jnp.float32)

    full = lambda s: pl.BlockSpec(s, lambda i: tuple(0 for _ in s))
    row = lambda w: pl.BlockSpec((B, w), lambda i: (i, 0))
    pspec = pl.BlockSpec((2, B, 4 * LATENT), lambda i: (0, i, 0))
    in_specs = [pspec, row(LATENT), row(LATENT), row(LATENT), row(LATENT),
                full((4 * LATENT, LATENT)),
                full((LATENT, LATENT)), full((LATENT, LATENT)), full((LATENT, LATENT)),
                full((1, LATENT)), full((1, LATENT)), full((1, LATENT))]
    if final:
        in_specs += [full((LATENT, LATENT)), full((1, LATENT)),
                     full((LATENT, LATENT)), full((1, LATENT))]
        out_specs = [row(LATENT), row(LATENT)]
        out_shape = [jax.ShapeDtypeStruct((N_NODES, LATENT), jnp.float32)] * 2
    else:
        out_specs = [row(LATENT)]
        out_shape = [jax.ShapeDtypeStruct((N_NODES, LATENT), jnp.float32)]
    return pl.pallas_call(
        body, grid=(nb,), in_specs=in_specs, out_specs=out_specs,
        out_shape=out_shape)


def kernel(x, edge_index, edge_attr, W_lin, b_lin, gnn_weight,
           gru_w_ih, gru_w_hh, gru_b_ih, gru_b_hh,
           W_mu, b_mu, W_lv, b_lv):
    ei = edge_index.astype(jnp.int32)
    src = jnp.pad(ei[0], (0, EP - N_EDGES)).reshape(EP // GW, GW)
    dst = jnp.pad(ei[1], (0, EP - N_EDGES),
                  constant_values=N_NODES).reshape(EP // GW, GW)
    acol = [jnp.pad(edge_attr[:, k], (0, EP - N_EDGES)).reshape(EP // GW, GW)
            for k in range(4)]

    row1 = lambda v: v.reshape(1, LATENT)
    wl = W_lin.T
    whh = [gru_w_hh[k * LATENT:(k + 1) * LATENT].T for k in range(3)]
    bhh = [row1(gru_b_hh[k * LATENT:(k + 1) * LATENT]) for k in range(3)]
    wih = [gru_w_ih[k * LATENT:(k + 1) * LATENT].T for k in range(3)]
    bih = [row1(gru_b_ih[k * LATENT:(k + 1) * LATENT]) for k in range(3)]
    wcat = [2.0 * gnn_weight[i].astype(jnp.bfloat16).astype(jnp.float32)
            .transpose(0, 2, 1).reshape(4 * LATENT, LATENT)
            for i in range(gnn_weight.shape[0])]

    nb = 10
    h0, h0b, ghr, ghz, ghn, comb = _tc_input_proj(nb)(
        x, wl, row1(b_lin), whh[0], whh[1], whh[2], bhh[0], bhh[1], bhh[2],
        acol[0], acol[1], acol[2], acol[3], dst)

    sc = _sc_gather_scatter()
    gru = _tc_gru(nb, final=False)
    gru_final = _tc_gru(nb, final=True)

    m = h0b
    num_layers = gnn_weight.shape[0]
    for i in range(num_layers):
        P = sc(m, src, comb)
        pm = P.reshape(2, BROWS, 4 * LATENT)
        args = (pm, h0, ghr, ghz, ghn, wcat[i],
                wih[0], wih[1], wih[2], bih[0], bih[1], bih[2])
        if i == num_layers - 1:
            mu, lv = gru_final(*args, W_mu.T, row1(b_mu), W_lv.T, row1(b_lv))
        else:
            (m,) = gru(*args)
    return (mu, lv)


# speedup vs baseline: 1.4508x; 1.0660x over previous
import functools

import jax
import jax.numpy as jnp
from jax import lax
from jax.experimental import pallas as pl
from jax.experimental.pallas import tpu as pltpu
from jax.experimental.pallas import tpu_sc as plsc

N_NODES = 10000
LATENT = 32
N_EDGES = 160000
NW = 32
GW = 128
GROUPS = 40
EP = NW * GROUPS * GW
BROWS = 10016
SPAD = 4 * BROWS
ZROWS = 128


def _sc_gather_scatter():
    mesh = plsc.VectorSubcoreMesh(core_axis_name="c", subcore_axis_name="s")

    @functools.partial(
        pl.kernel,
        mesh=mesh,
        compiler_params=pltpu.CompilerParams(use_tc_tiling_on_sc=False),
        out_type=jax.ShapeDtypeStruct((2, SPAD, LATENT), jnp.float32),
        scratch_types=[
            pltpu.VMEM((GROUPS, GW), jnp.int32),
            pltpu.VMEM((GROUPS, GW), jnp.int32),
            pltpu.VMEM((8, GW, LATENT), jnp.float32),
            pltpu.VMEM((ZROWS, LATENT), jnp.float32),
            pltpu.VMEM_SHARED((SPAD, LATENT), jnp.float32),
            [pltpu.SemaphoreType.DMA] * 8,
            [pltpu.SemaphoreType.DMA] * 8,
        ],
    )
    def sc_fn(table, src, comb, out, src_v, comb_v,
              rows, zbuf, acc, gsem, ssem):
        cid = lax.axis_index("c")
        sid = lax.axis_index("s")
        wid = sid * 2 + cid

        pltpu.sync_copy(src.at[pl.ds(wid * GROUPS, GROUPS)], src_v)
        pltpu.sync_copy(comb.at[pl.ds(wid * GROUPS, GROUPS)], comb_v)

        z16 = jnp.zeros((16,), jnp.float32)

        def zrow(r, carry):
            zbuf[r, pl.ds(0, 16)] = z16
            zbuf[r, pl.ds(16, 16)] = z16
            return carry

        lax.fori_loop(0, ZROWS, zrow, 0, unroll=8)
        for k in range(19):
            pltpu.sync_copy(zbuf, acc.at[pl.ds(sid * 2504 + k * ZROWS, ZROWS)])
        pltpu.sync_copy(zbuf.at[pl.ds(0, 72)],
                        acc.at[pl.ds(sid * 2504 + 2432, 72)])

        plsc.subcore_barrier()

        ND = 8

        def body(g, carry):
            for b in range(ND):
                @pl.when(g + b >= ND)
                def _():
                    pltpu.make_async_copy(
                        rows.at[b], acc.at[comb_v.at[g + b - ND]],
                        ssem[b]).wait()
                pltpu.async_copy(table.at[src_v.at[g + b]], rows.at[b], gsem[b])
            for b in range(ND):
                pltpu.make_async_copy(table.at[src_v.at[g + b]], rows.at[b],
                                      gsem[b]).wait()
                pltpu.async_copy(rows.at[b], acc.at[comb_v.at[g + b]], ssem[b],
                                 add=True)
            return carry

        lax.fori_loop(0, GROUPS // ND, lambda i, c: body(i * ND, c), 0)
        for b in range(ND):
            pltpu.make_async_copy(rows.at[b], acc.at[comb_v.at[GROUPS - ND + b]],
                                  ssem[b]).wait()

        plsc.subcore_barrier()
        pltpu.sync_copy(acc.at[pl.ds(sid * 2504, 2504)],
                        out.at[cid, pl.ds(sid * 2504, 2504)])

    return sc_fn


def _tc_input_proj(nb):
    B = N_NODES // nb
    GB = (EP // GW) // nb

    def body(x_r, wl_r, bl_r, wr_r, wz_r, wn_r, br_r, bz_r, bn_r,
             a0_r, a1_r, a2_r, a3_r, dst_r,
             h0_r, h0b_r, ghr_r, ghz_r, ghn_r, comb_r):
        h0 = jnp.maximum(
            jnp.dot(x_r[...], wl_r[...], preferred_element_type=jnp.float32)
            + bl_r[...], 0.0)
        h0_r[...] = h0
        h0b_r[...] = h0.astype(jnp.bfloat16).astype(jnp.float32)
        ghr_r[...] = jnp.dot(h0, wr_r[...], preferred_element_type=jnp.float32) + br_r[...]
        ghz_r[...] = jnp.dot(h0, wz_r[...], preferred_element_type=jnp.float32) + bz_r[...]
        ghn_r[...] = jnp.dot(h0, wn_r[...], preferred_element_type=jnp.float32) + bn_r[...]
        best = a0_r[...]
        b = jnp.zeros_like(dst_r[...])
        for k, ak_r in ((1, a1_r), (2, a2_r), (3, a3_r)):
            ak = ak_r[...]
            gt = ak > best
            best = jnp.where(gt, ak, best)
            b = jnp.where(gt, k, b)
        comb_r[...] = dst_r[...] * 4 + b

    full = lambda s: pl.BlockSpec(s, lambda i: tuple(0 for _ in s))
    row = lambda w: pl.BlockSpec((B, w), lambda i: (i, 0))
    erow = pl.BlockSpec((GB, GW), lambda i: (i, 0))
    return pl.pallas_call(
        body,
        grid=(nb,),
        in_specs=[row(128), full((128, LATENT)), full((1, LATENT)),
                  full((LATENT, LATENT)), full((LATENT, LATENT)), full((LATENT, LATENT)),
                  full((1, LATENT)), full((1, LATENT)), full((1, LATENT)),
                  erow, erow, erow, erow, erow],
        out_specs=[row(LATENT)] * 5 + [erow],
        out_shape=[jax.ShapeDtypeStruct((N_NODES, LATENT), jnp.float32)] * 5
        + [jax.ShapeDtypeStruct((EP // GW, GW), jnp.int32)],
    )


def _tc_gru(nb, final):
    B = N_NODES // nb

    def body(*refs):
        (p_r, h0_r, ghr_r, ghz_r, ghn_r, wcat_r,
         wir_r, wiz_r, win_r, bir_r, biz_r, bin_r) = refs[:12]
        if final:
            wmu_r, bmu_r, wlv_r, blv_r = refs[12:16]
            outs = refs[16:]
        else:
            outs = refs[12:]
        agg = jnp.dot(p_r[0] + p_r[1], wcat_r[...],
                      preferred_element_type=jnp.float32,
                      precision=jax.lax.Precision.HIGHEST)
        gir = jnp.dot(agg, wir_r[...], preferred_element_type=jnp.float32) + bir_r[...]
        giz = jnp.dot(agg, wiz_r[...], preferred_element_type=jnp.float32) + biz_r[...]
        gin = jnp.dot(agg, win_r[...], preferred_element_type=jnp.float32) + bin_r[...]
        r = jax.nn.sigmoid(gir + ghr_r[...])
        z = jax.nn.sigmoid(giz + ghz_r[...])
        n = jnp.tanh(gin + r * ghn_r[...])
        h0 = h0_r[...]
        m = jnp.maximum((1.0 - z) * n + z * h0, 0.0)
        if final:
            outs[0][...] = jnp.dot(m, wmu_r[...], preferred_element_type=jnp.float32) + bmu_r[...]
            outs[1][...] = jnp.dot(m, wlv_r[...], preferred_element_type=jnp.float32) + blv_r[...]
        else:
            outs[0][...] = m.astype(jnp.bfloat16).astype(jnp.float32)

    full = lambda s: pl.BlockSpec(s, lambda i: tuple(0 for _ in s))
    row = lambda w: pl.BlockSpec((B, w), lambda i: (i, 0))
    pspec = pl.BlockSpec((2, B, 4 * LATENT), lambda i: (0, i, 0))
    in_specs = [pspec, row(LATENT), row(LATENT), row(LATENT), row(LATENT),
                full((4 * LATENT, LATENT)),
                full((LATENT, LATENT)), full((LATENT, LATENT)), full((LATENT, LATENT)),
                full((1, LATENT)), full((1, LATENT)), full((1, LATENT))]
    if final:
        in_specs += [full((LATENT, LATENT)), full((1, LATENT)),
                     full((LATENT, LATENT)), full((1, LATENT))]
        out_specs = [row(LATENT), row(LATENT)]
        out_shape = [jax.ShapeDtypeStruct((N_NODES, LATENT), jnp.float32)] * 2
    else:
        out_specs = [row(LATENT)]
        out_shape = [jax.ShapeDtypeStruct((N_NODES, LATENT), jnp.float32)]
    return pl.pallas_call(
        body, grid=(nb,), in_specs=in_specs, out_specs=out_specs,
        out_shape=out_shape)


def kernel(x, edge_index, edge_attr, W_lin, b_lin, gnn_weight,
           gru_w_ih, gru_w_hh, gru_b_ih, gru_b_hh,
           W_mu, b_mu, W_lv, b_lv):
    ei = edge_index.astype(jnp.int32)
    src = jnp.pad(ei[0], (0, EP - N_EDGES)).reshape(EP // GW, GW)
    dst = jnp.pad(ei[1], (0, EP - N_EDGES),
                  constant_values=N_NODES).reshape(EP // GW, GW)
    acol = [jnp.pad(edge_attr[:, k], (0, EP - N_EDGES)).reshape(EP // GW, GW)
            for k in range(4)]

    row1 = lambda v: v.reshape(1, LATENT)
    wl = W_lin.T
    whh = [gru_w_hh[k * LATENT:(k + 1) * LATENT].T for k in range(3)]
    bhh = [row1(gru_b_hh[k * LATENT:(k + 1) * LATENT]) for k in range(3)]
    wih = [gru_w_ih[k * LATENT:(k + 1) * LATENT].T for k in range(3)]
    bih = [row1(gru_b_ih[k * LATENT:(k + 1) * LATENT]) for k in range(3)]
    wcat = [2.0 * gnn_weight[i].astype(jnp.bfloat16).astype(jnp.float32)
            .transpose(0, 2, 1).reshape(4 * LATENT, LATENT)
            for i in range(gnn_weight.shape[0])]

    nb = 5
    h0, h0b, ghr, ghz, ghn, comb = _tc_input_proj(nb)(
        x, wl, row1(b_lin), whh[0], whh[1], whh[2], bhh[0], bhh[1], bhh[2],
        acol[0], acol[1], acol[2], acol[3], dst)

    sc = _sc_gather_scatter()
    gru = _tc_gru(nb, final=False)
    gru_final = _tc_gru(nb, final=True)

    m = h0b
    num_layers = gnn_weight.shape[0]
    for i in range(num_layers):
        P = sc(m, src, comb)
        pm = P.reshape(2, BROWS, 4 * LATENT)
        args = (pm, h0, ghr, ghz, ghn, wcat[i],
                wih[0], wih[1], wih[2], bih[0], bih[1], bih[2])
        if i == num_layers - 1:
            mu, lv = gru_final(*args, W_mu.T, row1(b_mu), W_lv.T, row1(b_lv))
        else:
            (m,) = gru(*args)
    return (mu, lv)
